# baseline (speedup 1.0000x reference)
import functools

import jax
import jax.numpy as jnp
from jax import lax
from jax.experimental import pallas as pl
from jax.experimental.pallas import tpu as pltpu, tpu_sc as plsc

D = 64
B = 4096 * 200
SCALE = 8.0

_info = plsc.get_sparse_core_info()
NC, NS, L = _info.num_cores, _info.num_subcores, _info.num_lanes
NW = NC * NS
B_PER_W = B // NW
CB = 512
N_CHUNKS = B_PER_W // CB


def _sc_embed(x_flat, lut):
    mesh = plsc.VectorSubcoreMesh(core_axis_name="c", subcore_axis_name="s")

    @functools.partial(
        pl.kernel,
        mesh=mesh,
        compiler_params=pltpu.CompilerParams(use_tc_tiling_on_sc=False),
        out_type=jax.ShapeDtypeStruct((B, D), jnp.float32),
        scratch_types=[
            pltpu.VMEM((CB,), jnp.int32),
            pltpu.VMEM((CB, D), jnp.float32),
            pltpu.SemaphoreType.DMA,
        ],
    )
    def k(idx_hbm, table_hbm, out_hbm, idx_v, rows_v, sem):
        wid = lax.axis_index("s") * NC + lax.axis_index("c")
        base = wid * B_PER_W

        def chunk_body(g, carry):
            off = base + g * CB
            pltpu.sync_copy(idx_hbm.at[pl.ds(off, CB)], idx_v)
            pltpu.async_copy(table_hbm.at[idx_v], rows_v, sem).wait()

            def scale_row(r, c2):
                for c4 in range(D // L):
                    sl = pl.ds(c4 * L, L)
                    rows_v[r, sl] = rows_v[r, sl] * SCALE
                return c2

            lax.fori_loop(0, CB, scale_row, 0, unroll=4)
            pltpu.sync_copy(rows_v, out_hbm.at[pl.ds(off, CB)])
            return carry

        lax.fori_loop(0, N_CHUNKS, chunk_body, 0)

    return k(x_flat, lut)


def kernel(x, lut):
    x_flat = x.reshape(-1).astype(jnp.int32)
    out = _sc_embed(x_flat, lut)
    return out.reshape(x.shape[0], x.shape[1], D)


# speedup vs baseline: 1.0888x; 1.0888x over previous
import functools

import jax
import jax.numpy as jnp
from jax import lax
from jax.experimental import pallas as pl
from jax.experimental.pallas import tpu as pltpu, tpu_sc as plsc

D = 64
B = 4096 * 200
SCALE = 8.0

_info = plsc.get_sparse_core_info()
NC, NS, L = _info.num_cores, _info.num_subcores, _info.num_lanes
NW = NC * NS
B_PER_W = B // NW
CB = 320
NBUF = 4
N_CHUNKS = B_PER_W // CB


def _sc_embed(x_flat, lut):
    mesh = plsc.VectorSubcoreMesh(core_axis_name="c", subcore_axis_name="s")

    @functools.partial(
        pl.kernel,
        mesh=mesh,
        compiler_params=pltpu.CompilerParams(use_tc_tiling_on_sc=False),
        out_type=jax.ShapeDtypeStruct((B, D), jnp.float32),
        scratch_types=(
            [pltpu.VMEM((B_PER_W,), jnp.int32)]
            + [pltpu.VMEM((CB, D), jnp.float32) for _ in range(NBUF)]
            + [pltpu.SemaphoreType.DMA for _ in range(2 * NBUF)]
        ),
    )
    def k(idx_hbm, table_hbm, out_hbm, idx_all, *bufs_and_sems):
        rows = bufs_and_sems[:NBUF]
        sg = bufs_and_sems[NBUF:2 * NBUF]
        sw = bufs_and_sems[2 * NBUF:3 * NBUF]

        wid = lax.axis_index("s") * NC + lax.axis_index("c")
        base = wid * B_PER_W
        pltpu.sync_copy(idx_hbm.at[pl.ds(base, B_PER_W)], idx_all)

        def gather_start(g, b):
            idx_sl = idx_all.at[pl.ds(g * CB, CB)]
            pltpu.async_copy(table_hbm.at[idx_sl], rows[b], sg[b])

        def gather_wait(g, b):
            idx_sl = idx_all.at[pl.ds(g * CB, CB)]
            pltpu.make_async_copy(table_hbm.at[idx_sl], rows[b], sg[b]).wait()

        def wb_start(g, b):
            pltpu.async_copy(rows[b], out_hbm.at[pl.ds(base + g * CB, CB)],
                             sw[b])

        def wb_wait(b):
            pltpu.make_async_copy(rows[b], out_hbm.at[pl.ds(base, CB)],
                                  sw[b]).wait()

        def scale_chunk(b):
            def scale_row(r, c2):
                for c4 in range(D // L):
                    sl = pl.ds(c4 * L, L)
                    rows[b][r, sl] = rows[b][r, sl] * SCALE
                return c2
            lax.fori_loop(0, CB, scale_row, 0, unroll=8)

        for b in range(NBUF - 1):
            gather_start(b, b)

        def outer(go, carry):
            for b in range(NBUF):
                g = go * NBUF + b
                nb = (b + NBUF - 1) % NBUF

                @pl.when(jnp.logical_and(g + NBUF - 1 < N_CHUNKS, g >= 1))
                def _():
                    wb_wait(nb)

                @pl.when(g + NBUF - 1 < N_CHUNKS)
                def _():
                    gather_start(g + NBUF - 1, nb)

                gather_wait(g, b)
                scale_chunk(b)
                wb_start(g, b)
            return carry

        lax.fori_loop(0, N_CHUNKS // NBUF, outer, 0)
        for b in range(NBUF):
            wb_wait(b)

    return k(x_flat, lut)


def kernel(x, lut):
    x_flat = x.reshape(-1).astype(jnp.int32)
    out = _sc_embed(x_flat, lut)
    return out.reshape(x.shape[0], x.shape[1], D)
